# Initial kernel scaffold; baseline (speedup 1.0000x reference)
import jax
import jax.numpy as jnp
from jax.experimental import pallas as pl


def kernel(x, t, W_time, b_time, W0, a_src0, a_dst0, Ws0, bias0, W1, a_src1, a_dst1, Ws1, bias1, W2, a_src2, a_dst2, bias2, W_nh1, b_nh1, W_nh2, b_nh2):
    raise NotImplementedError("write your pallas kernel here")


# speedup vs baseline: 152.0947x; 152.0947x over previous
import functools

import jax
import jax.numpy as jnp
import numpy as np
from jax.experimental import pallas as pl

B = 1024
V = 64
TDIM = 128
G = 32


def _dot(a, b):
    return jax.lax.dot_general(a, b, (((1,), (0,)), ((), ())),
                               preferred_element_type=jnp.float32)


def _leaky(x):
    return jnp.where(x >= 0, x, 0.2 * x)


def _silu(x):
    return x * jax.lax.logistic(x)


def _elu(x):
    return jnp.where(x > 0, x, jnp.exp(jnp.minimum(x, 0.0)) - 1.0)


def _roll_prev(x3):
    return jnp.concatenate([x3[:, V - 1:, :], x3[:, :V - 1, :]], axis=1)


def _roll_next(x3):
    return jnp.concatenate([x3[:, 1:, :], x3[:, :1, :]], axis=1)


def _gat(proj, skip, A, bias, nh, E, act):
    GV = proj.shape[0]
    g = GV // V
    sa = _dot(proj, A)
    ss = sa[:, :nh].reshape(g, V, nh)
    st = sa[:, nh:].reshape(g, V, nh)
    e_s = _leaky(ss + st)
    e_p = _leaky(_roll_prev(ss) + st)
    e_n = _leaky(_roll_next(ss) + st)
    m = jnp.maximum(e_s, jnp.maximum(e_p, e_n))
    x_s = jnp.exp(e_s - m)
    x_p = jnp.exp(e_p - m)
    x_n = jnp.exp(e_n - m)
    den = x_s + x_p + x_n + 1e-16
    w_s = (x_s / den).reshape(GV, nh)
    w_p = (x_p / den).reshape(GV, nh)
    w_n = (x_n / den).reshape(GV, nh)
    if nh > 1:
        w_s = _dot(w_s, E)
        w_p = _dot(w_p, E)
        w_n = _dot(w_n, E)
    proj3 = proj.reshape(g, V, 256)
    p_p = _roll_prev(proj3).reshape(GV, 256)
    p_n = _roll_next(proj3).reshape(GV, 256)
    out = w_s * proj + w_p * p_p + w_n * p_n
    out = out + skip + bias
    return _elu(out) if act else out


def _body(x2_ref, tf_ref, freqs_ref, pos_ref, Wt_ref, bt_ref,
          W0c_ref, W0p_ref, W0t_ref, Ws0c_ref, Ws0p_ref, Ws0t_ref,
          A0_ref, b0_ref, W1_ref, Ws1_ref, A1_ref, b1_ref,
          W2_ref, A2_ref, b2_ref, Wn1_ref, bn1_ref, Wn2_ref, bn2_ref,
          E_ref, out_ref):
    GV = G * V
    coords = x2_ref[...]
    tf = tf_ref[...]
    ang = tf * freqs_ref[...]
    sincos = jnp.concatenate([jnp.sin(ang), jnp.cos(ang)], axis=1)
    temb = _silu(_dot(sincos, Wt_ref[...]) + bt_ref[...])
    pos = pos_ref[...]

    def in_proj(Wc, Wp, Wt128):
        c = coords[:, 0:1] * Wc[0:1, :] + coords[:, 1:2] * Wc[1:2, :]
        p = _dot(pos, Wp)
        tm = _dot(temb, Wt128)
        h3 = c.reshape(G, V, 256) + p[None, :, :] + tm[:, None, :]
        return h3.reshape(GV, 256)

    proj0 = in_proj(W0c_ref[...], W0p_ref[...], W0t_ref[...])
    skip0 = in_proj(Ws0c_ref[...], Ws0p_ref[...], Ws0t_ref[...])
    E = E_ref[...]
    h1 = _gat(proj0, skip0, A0_ref[...], b0_ref[...], 4, E, True)
    h2 = _gat(_dot(h1, W1_ref[...]), _dot(h1, Ws1_ref[...]),
              A1_ref[...], b1_ref[...], 4, E, True)
    h3 = _gat(_dot(h2, W2_ref[...]), h2, A2_ref[...], b2_ref[...],
              1, None, False)
    hh = _silu(_dot(h3, Wn1_ref[...]) + bn1_ref[...])
    out_ref[...] = _dot(hh, Wn2_ref[...]) + bn2_ref[...]


def kernel(x, t, W_time, b_time, W0, a_src0, a_dst0, Ws0, bias0,
           W1, a_src1, a_dst1, Ws1, bias1, W2, a_src2, a_dst2, bias2,
           W_nh1, b_nh1, W_nh2, b_nh2):
    N = B * V
    GV = G * V
    x2 = x.reshape(N, 2)
    tf = t.astype(jnp.float32).reshape(B, 1)

    half = TDIM // 2
    freqs = jnp.exp(-jnp.log(10000.0)
                    * jnp.arange(half, dtype=jnp.float32) / (half - 1))
    freqs = freqs.reshape(1, half)
    phase = jnp.arange(V, dtype=jnp.float32) * (2.0 * np.pi / V)
    pos = jnp.stack([jnp.sin(phase), jnp.cos(phase),
                     jnp.sin(2.0 * phase), jnp.cos(2.0 * phase)], axis=1)

    E = jnp.repeat(jnp.eye(4, dtype=jnp.float32), 64, axis=1)

    def attn_mat(a_s, a_t, nh):
        if nh == 1:
            return jnp.concatenate([a_s.T, a_t.T], axis=1)
        As = (E * a_s.reshape(-1)[None, :]).T
        At = (E * a_t.reshape(-1)[None, :]).T
        return jnp.concatenate([As, At], axis=1)

    A0 = attn_mat(a_src0, a_dst0, 4)
    A1 = attn_mat(a_src1, a_dst1, 4)
    A2 = attn_mat(a_src2, a_dst2, 1)

    row = lambda i: (i, 0)
    rep = lambda i: (0, 0)
    in_specs = [
        pl.BlockSpec((GV, 2), row),
        pl.BlockSpec((G, 1), row),
        pl.BlockSpec((1, half), rep),
        pl.BlockSpec((V, 4), rep),
        pl.BlockSpec((TDIM, TDIM), rep),
        pl.BlockSpec((1, TDIM), rep),
        pl.BlockSpec((2, 256), rep),
        pl.BlockSpec((4, 256), rep),
        pl.BlockSpec((TDIM, 256), rep),
        pl.BlockSpec((2, 256), rep),
        pl.BlockSpec((4, 256), rep),
        pl.BlockSpec((TDIM, 256), rep),
        pl.BlockSpec((256, 8), rep),
        pl.BlockSpec((1, 256), rep),
        pl.BlockSpec((256, 256), rep),
        pl.BlockSpec((256, 256), rep),
        pl.BlockSpec((256, 8), rep),
        pl.BlockSpec((1, 256), rep),
        pl.BlockSpec((256, 256), rep),
        pl.BlockSpec((256, 2), rep),
        pl.BlockSpec((1, 256), rep),
        pl.BlockSpec((256, 256), rep),
        pl.BlockSpec((1, 256), rep),
        pl.BlockSpec((256, 2), rep),
        pl.BlockSpec((1, 2), rep),
        pl.BlockSpec((4, 256), rep),
    ]
    node = pl.pallas_call(
        _body,
        grid=(B // G,),
        in_specs=in_specs,
        out_specs=pl.BlockSpec((GV, 2), row),
        out_shape=jax.ShapeDtypeStruct((N, 2), jnp.float32),
    )(x2, tf, freqs, pos, W_time, b_time.reshape(1, TDIM),
      W0[0:2], W0[2:6], W0[6:134], Ws0[0:2], Ws0[2:6], Ws0[6:134],
      A0, bias0.reshape(1, 256),
      W1, Ws1, A1, bias1.reshape(1, 256),
      W2, A2, bias2.reshape(1, 256),
      W_nh1, b_nh1.reshape(1, 256), W_nh2, b_nh2.reshape(1, 2), E)
    return node.reshape(B, 2 * V)
